# baseline (speedup 1.0000x reference)
import functools

import jax
import jax.numpy as jnp
from jax import lax
from jax.experimental import pallas as pl
from jax.experimental.pallas import tpu as pltpu
from jax.experimental.pallas import tpu_sc as plsc

_NUM_EMBEDDINGS = 1000000
_DIM = 64
_BATCH = 16384
_HIST = 50
_B = _BATCH * _HIST
_NSPLIT = 2
_BS = _B // _NSPLIT
_NW = 32
_B_PER_W = _BS // _NW
_CHUNK = 320
_N_CHUNKS = _B_PER_W // _CHUNK
_NBUF = 4
_N_GROUPS = _N_CHUNKS // _NBUF

_mesh = plsc.VectorSubcoreMesh(core_axis_name="c", subcore_axis_name="s")


@functools.partial(
    pl.kernel,
    mesh=_mesh,
    out_type=jax.ShapeDtypeStruct((_BS, _DIM), jnp.float32),
    scratch_types=[
        [pltpu.VMEM((_CHUNK,), jnp.int32) for _ in range(_NBUF)],
        [pltpu.VMEM((_CHUNK, _DIM), jnp.float32) for _ in range(_NBUF)],
        [pltpu.SemaphoreType.DMA for _ in range(_NBUF)],
        [pltpu.SemaphoreType.DMA for _ in range(_NBUF)],
        [pltpu.SemaphoreType.DMA for _ in range(_NBUF)],
    ],
    compiler_params=pltpu.CompilerParams(use_tc_tiling_on_sc=False),
)
def _gather_rows(idx_hbm, table_hbm, out_hbm, idxs, bufs, isems, gsems, ssems):
    wid = lax.axis_index("s") * 2 + lax.axis_index("c")
    base = wid * _B_PER_W

    def i_copy(i, b):
        return pltpu.make_async_copy(
            idx_hbm.at[pl.ds(base + i * _CHUNK, _CHUNK)], idxs[b], isems[b])

    def g_copy(i, b):
        return pltpu.make_async_copy(table_hbm.at[idxs[b]], bufs[b], gsems[b])

    def s_copy(i, b):
        return pltpu.make_async_copy(
            bufs[b], out_hbm.at[pl.ds(base + i * _CHUNK, _CHUNK)], ssems[b])

    for b in range(_NBUF):
        i_copy(b, b).start()
    for b in range(2):
        i_copy(b, b).wait()
        g_copy(b, b).start()

    def step(i, k, do_sw, do_next, do_refill):
        g_copy(i, k).wait()
        s_copy(i, k).start()
        if do_next:
            if do_sw:
                s_copy(i - 2, (k - 2) % _NBUF).wait()
            i_copy(i + 2, (k + 2) % _NBUF).wait()
            g_copy(i + 2, (k + 2) % _NBUF).start()
        if do_refill:
            i_copy(i + _NBUF, k).start()

    for k in range(_NBUF):
        step(k, k, do_sw=(k >= 2), do_next=True, do_refill=True)

    def body(g, carry):
        i0 = g * _NBUF
        for k in range(_NBUF):
            step(i0 + k, k, do_sw=True, do_next=True, do_refill=True)
        return carry

    lax.fori_loop(1, _N_GROUPS - 1, body, 0)

    i0 = (_N_GROUPS - 1) * _NBUF
    for k in range(_NBUF):
        step(i0 + k, k, do_sw=(k < 2), do_next=(k < 2), do_refill=False)

    s_copy(_N_CHUNKS - 4, (_N_CHUNKS - 4) % _NBUF).wait()
    s_copy(_N_CHUNKS - 3, (_N_CHUNKS - 3) % _NBUF).wait()
    s_copy(_N_CHUNKS - 2, (_N_CHUNKS - 2) % _NBUF).wait()
    s_copy(_N_CHUNKS - 1, (_N_CHUNKS - 1) % _NBUF).wait()


def kernel(x, table):
    flat_idx = x.reshape(_B).astype(jnp.int32)
    halves = [
        _gather_rows(flat_idx[s * _BS:(s + 1) * _BS], table)
        for s in range(_NSPLIT)
    ]
    out = jnp.concatenate(halves, axis=0)
    return out.reshape(_BATCH, _HIST, _DIM)


# speedup vs baseline: 1.5060x; 1.5060x over previous
import functools

import jax
import jax.numpy as jnp
from jax import lax
from jax.experimental import pallas as pl
from jax.experimental.pallas import tpu as pltpu
from jax.experimental.pallas import tpu_sc as plsc

_NUM_EMBEDDINGS = 1000000
_DIM = 64
_BATCH = 16384
_HIST = 50
_B = _BATCH * _HIST
_NW = 32
_B_PER_W = _B // _NW
_CHUNK = 400
_N_CHUNKS = _B_PER_W // _CHUNK
_NBUF = 4
_N_GROUPS = _N_CHUNKS // _NBUF

_mesh = plsc.VectorSubcoreMesh(core_axis_name="c", subcore_axis_name="s")


@functools.partial(
    pl.kernel,
    mesh=_mesh,
    out_type=jax.ShapeDtypeStruct((_B, _DIM), jnp.float32),
    scratch_types=[
        [pltpu.VMEM((_CHUNK,), jnp.int32) for _ in range(_NBUF)],
        [pltpu.VMEM((_CHUNK, _DIM), jnp.float32) for _ in range(_NBUF)],
        [pltpu.SemaphoreType.DMA for _ in range(_NBUF)],
        [pltpu.SemaphoreType.DMA for _ in range(_NBUF)],
        [pltpu.SemaphoreType.DMA for _ in range(_NBUF)],
    ],
    compiler_params=pltpu.CompilerParams(use_tc_tiling_on_sc=False),
)
def _gather_rows(idx_hbm, table_hbm, out_hbm, idxs, bufs, isems, gsems, ssems):
    wid = lax.axis_index("s") * 2 + lax.axis_index("c")
    base = wid * _B_PER_W

    def i_copy(i, b):
        return pltpu.make_async_copy(
            idx_hbm.at[pl.ds(base + i * _CHUNK, _CHUNK)], idxs[b], isems[b])

    def g_copy(i, b):
        return pltpu.make_async_copy(table_hbm.at[idxs[b]], bufs[b], gsems[b])

    def s_copy(i, b):
        return pltpu.make_async_copy(
            bufs[b], out_hbm.at[pl.ds(base + i * _CHUNK, _CHUNK)], ssems[b])

    for b in range(_NBUF):
        i_copy(b, b).start()
    for b in range(2):
        i_copy(b, b).wait()
        g_copy(b, b).start()

    def step(i, k, do_sw, do_next, do_refill):
        g_copy(i, k).wait()
        s_copy(i, k).start()
        if do_next:
            if do_sw:
                s_copy(i - 2, (k - 2) % _NBUF).wait()
            i_copy(i + 2, (k + 2) % _NBUF).wait()
            g_copy(i + 2, (k + 2) % _NBUF).start()
        if do_refill:
            i_copy(i + _NBUF, k).start()

    for k in range(_NBUF):
        step(k, k, do_sw=(k >= 2), do_next=True, do_refill=True)

    def body(g, carry):
        i0 = g * _NBUF
        for k in range(_NBUF):
            step(i0 + k, k, do_sw=True, do_next=True, do_refill=True)
        return carry

    lax.fori_loop(1, _N_GROUPS - 1, body, 0)

    i0 = (_N_GROUPS - 1) * _NBUF
    for k in range(_NBUF):
        step(i0 + k, k, do_sw=(k < 2), do_next=(k < 2), do_refill=False)

    s_copy(_N_CHUNKS - 4, (_N_CHUNKS - 4) % _NBUF).wait()
    s_copy(_N_CHUNKS - 3, (_N_CHUNKS - 3) % _NBUF).wait()
    s_copy(_N_CHUNKS - 2, (_N_CHUNKS - 2) % _NBUF).wait()
    s_copy(_N_CHUNKS - 1, (_N_CHUNKS - 1) % _NBUF).wait()


def kernel(x, table):
    flat_idx = x.reshape(_B).astype(jnp.int32)
    out = _gather_rows(flat_idx, table)
    return out.reshape(_BATCH, _HIST, _DIM)
